# baseline (speedup 1.0000x reference)
import functools

import jax
import jax.numpy as jnp
from jax import lax
from jax.experimental import pallas as pl
from jax.experimental.pallas import tpu as pltpu
from jax.experimental.pallas import tpu_sc as plsc

N = 10000
E = 320000
D = 128
DH = D // 2
DO = 16
NN = 10016
NC = 2
NS = 16
NW = NC * NS
CHUNK = 128
CPT = 80
CPT2 = 160
EP = NW * CPT * CHUNK
TRASH = N
RPT = NN // NS
NBUF = 5
G = 5
R = N // G

_mesh = plsc.VectorSubcoreMesh(core_axis_name="c", subcore_axis_name="s")


@functools.partial(
    pl.kernel,
    out_type=jax.ShapeDtypeStruct((NC, NN, DO), jnp.float32),
    mesh=_mesh,
    scratch_types=[
        pltpu.VMEM((CPT, CHUNK), jnp.int32),
        pltpu.VMEM((CHUNK, DO), jnp.float32),
        pltpu.SemaphoreType.DMA,
        pltpu.SemaphoreType.DMA,
        pltpu.SemaphoreType.DMA,
        pltpu.SemaphoreType.DMA,
        pltpu.VMEM_SHARED((NN, DO), jnp.float32),
    ],
    compiler_params=pltpu.CompilerParams(use_tc_tiling_on_sc=False),
)
def _deg_kernel(dsts_hbm, ones_hbm, zeros_hbm, hist_out,
                dst_v, ones_v, s0, s1, s2, s3, hist_s):
    cid = lax.axis_index("c")
    sid = lax.axis_index("s")
    wid = cid * NS + sid
    ssem = (s0, s1, s2, s3)
    pltpu.sync_copy(dsts_hbm.at[wid], dst_v)
    pltpu.sync_copy(ones_hbm, ones_v)
    pltpu.sync_copy(zeros_hbm, hist_s.at[pl.ds(sid * RPT, RPT)])
    plsc.subcore_barrier()

    def body(i, carry):
        for b in range(4):
            j = 4 * i + b

            @pl.when(i > 0)
            def _():
                pltpu.make_async_copy(ones_v, hist_s.at[dst_v.at[j - 4]],
                                      ssem[b]).wait()

            pltpu.async_copy(ones_v, hist_s.at[dst_v.at[j]], ssem[b],
                             add=True)
        return carry

    lax.fori_loop(0, CPT // 4, body, 0)
    for b in range(4):
        pltpu.make_async_copy(ones_v, hist_s.at[dst_v.at[CPT - 4 + b]],
                              ssem[b]).wait()
    plsc.subcore_barrier()
    pltpu.sync_copy(hist_s.at[pl.ds(sid * RPT, RPT)],
                    hist_out.at[cid].at[pl.ds(sid * RPT, RPT)])


def _pipeline(table, src_v, dst_v, bufs, gsem, ssem, acc, n_chunks):
    def start_gather(j, b):
        pltpu.async_copy(table.at[src_v.at[j]], bufs[b], gsem[b])

    def wait_gather(j, b):
        pltpu.make_async_copy(table.at[src_v.at[j]], bufs[b], gsem[b]).wait()

    def start_scatter(j, b):
        pltpu.async_copy(bufs[b], acc.at[dst_v.at[j]], ssem[b], add=True)

    def wait_scatter(j, b):
        pltpu.make_async_copy(bufs[b], acc.at[dst_v.at[j]], ssem[b]).wait()

    ngroups = n_chunks // NBUF
    for b in range(NBUF):
        start_gather(b, b)

    def body(g, carry):
        c0 = NBUF * g
        for b in range(NBUF):
            wait_gather(c0 + b, b)
            start_scatter(c0 + b, b)
        for b in range(NBUF):
            wait_scatter(c0 + b, b)

            @pl.when(g < ngroups - 1)
            def _():
                start_gather(c0 + NBUF + b, b)

        return carry

    lax.fori_loop(0, ngroups, body, 0)


def _agg_scratch(cpt, d):
    return (
        [pltpu.VMEM((cpt, CHUNK), jnp.int32)] * 2
        + [pltpu.VMEM((CHUNK, d), jnp.float32)] * NBUF
        + [pltpu.SemaphoreType.DMA] * (2 * NBUF)
        + [pltpu.VMEM_SHARED((NN, d), jnp.float32)]
    )


_LASTB = N - (NS - 1) * RPT
_NGRP = CPT2 // NBUF
_NHALF = _NGRP // 2


@functools.partial(
    pl.kernel,
    out_type=jax.ShapeDtypeStruct((NC, NN, DH), jnp.float32),
    mesh=_mesh,
    scratch_types=(
        [pltpu.VMEM((NBUF, CHUNK), jnp.int32)] * 4
        + [pltpu.VMEM((CHUNK, DH), jnp.float32)] * NBUF
        + [pltpu.SemaphoreType.DMA] * (2 * NBUF + 4)
        + [pltpu.VMEM_SHARED((NN, DH), jnp.float32)]
        + [pltpu.VMEM_SHARED((NN, DH), jnp.float32)]
    ),
    compiler_params=pltpu.CompilerParams(use_tc_tiling_on_sc=False),
)
def _agg128(hs2_hbm, srcs_hbm, dsts_hbm, zeros_hbm, out_hbm,
            w0s, w0d, w1s, w1d, *rest):
    bufs = rest[:NBUF]
    gsem = rest[NBUF:2 * NBUF]
    ssem = rest[2 * NBUF:3 * NBUF]
    is0, id0, is1, id1 = rest[3 * NBUF:3 * NBUF + 4]
    table_s, acc = rest[-2], rest[-1]
    cid = lax.axis_index("c")
    sid = lax.axis_index("s")
    my_src = srcs_hbm.at[sid]
    my_dst = dsts_hbm.at[sid]

    def swin(g):
        return my_src.at[pl.ds(g * NBUF, NBUF)]

    def dwin(g):
        return my_dst.at[pl.ds(g * NBUF, NBUF)]

    def phase1(cs, cd, g):
        for b in range(NBUF):
            pltpu.make_async_copy(table_s.at[cs.at[b]], bufs[b],
                                  gsem[b]).wait()
            pltpu.async_copy(bufs[b], acc.at[cd.at[b]], ssem[b], add=True)

    def phase2(cd, ns, g):
        for b in range(NBUF):
            pltpu.make_async_copy(bufs[b], acc.at[cd.at[b]], ssem[b]).wait()

            @pl.when(g < _NGRP - 1)
            def _():
                pltpu.async_copy(table_s.at[ns.at[b]], bufs[b], gsem[b])

    pltpu.sync_copy(zeros_hbm, acc.at[pl.ds(sid * RPT, RPT)])

    @pl.when(sid < NS - 1)
    def _():
        pltpu.sync_copy(hs2_hbm.at[cid].at[pl.ds(sid * RPT, RPT)],
                        table_s.at[pl.ds(sid * RPT, RPT)])

    @pl.when(sid == NS - 1)
    def _():
        pltpu.sync_copy(hs2_hbm.at[cid].at[pl.ds((NS - 1) * RPT, _LASTB)],
                        table_s.at[pl.ds((NS - 1) * RPT, _LASTB)])

    pltpu.sync_copy(swin(0), w0s)
    pltpu.sync_copy(dwin(0), w0d)
    pltpu.async_copy(swin(1), w1s, is1)
    pltpu.async_copy(dwin(1), w1d, id1)
    plsc.subcore_barrier()
    for b in range(NBUF):
        pltpu.async_copy(table_s.at[w0s.at[b]], bufs[b], gsem[b])

    def body(i, carry):
        g0 = 2 * i
        g1 = g0 + 1
        pltpu.make_async_copy(swin(g1), w1s, is1).wait()
        pltpu.make_async_copy(dwin(g1), w1d, id1).wait()
        phase1(w0s, w0d, g0)

        @pl.when(i < _NHALF - 1)
        def _():
            pltpu.async_copy(swin(g0 + 2), w0s, is0)

        phase2(w0d, w1s, g0)

        @pl.when(i < _NHALF - 1)
        def _():
            pltpu.async_copy(dwin(g0 + 2), w0d, id0)
            pltpu.make_async_copy(swin(g0 + 2), w0s, is0).wait()

        phase1(w1s, w1d, g1)

        @pl.when(i < _NHALF - 1)
        def _():
            pltpu.async_copy(swin(g1 + 2), w1s, is1)

        phase2(w1d, w0s, g1)

        @pl.when(i < _NHALF - 1)
        def _():
            pltpu.async_copy(dwin(g1 + 2), w1d, id1)
            pltpu.make_async_copy(dwin(g0 + 2), w0d, id0).wait()

        return carry

    lax.fori_loop(0, _NHALF, body, 0)
    plsc.subcore_barrier()
    pltpu.sync_copy(acc.at[pl.ds(sid * RPT, RPT)],
                    out_hbm.at[cid].at[pl.ds(sid * RPT, RPT)])


_LAST = N - (NS - 1) * RPT


@functools.partial(
    pl.kernel,
    out_type=jax.ShapeDtypeStruct((NC, NN, DO), jnp.float32),
    mesh=_mesh,
    scratch_types=_agg_scratch(CPT, DO)
    + [pltpu.VMEM_SHARED((NN, DO), jnp.float32)],
    compiler_params=pltpu.CompilerParams(use_tc_tiling_on_sc=False),
)
def _agg16(h2s_hbm, srcs_hbm, dsts_hbm, zeros_hbm, out_hbm,
           src_v, dst_v, *rest):
    bufs, gsem, ssem, acc, table_s = (rest[:NBUF], rest[NBUF:2 * NBUF],
                                      rest[2 * NBUF:3 * NBUF], rest[-2],
                                      rest[-1])
    cid = lax.axis_index("c")
    sid = lax.axis_index("s")
    wid = cid * NS + sid
    pltpu.sync_copy(srcs_hbm.at[wid], src_v)
    pltpu.sync_copy(dsts_hbm.at[wid], dst_v)
    pltpu.sync_copy(zeros_hbm, acc.at[pl.ds(sid * RPT, RPT)])

    @pl.when(sid < NS - 1)
    def _():
        pltpu.sync_copy(h2s_hbm.at[pl.ds(sid * RPT, RPT)],
                        table_s.at[pl.ds(sid * RPT, RPT)])

    @pl.when(sid == NS - 1)
    def _():
        pltpu.sync_copy(h2s_hbm.at[pl.ds((NS - 1) * RPT, _LAST)],
                        table_s.at[pl.ds((NS - 1) * RPT, _LAST)])

    plsc.subcore_barrier()
    _pipeline(table_s, src_v, dst_v, bufs, gsem, ssem, acc, CPT)
    plsc.subcore_barrier()
    pltpu.sync_copy(acc.at[pl.ds(sid * RPT, RPT)],
                    out_hbm.at[cid].at[pl.ds(sid * RPT, RPT)])


def _tc1_body(x_ref, w1_ref, hist_ref, hs_ref, dinv_ref):
    deg = hist_ref[0, :, 0:1] + hist_ref[1, :, 0:1]
    dinv = lax.rsqrt(deg + 1.0)
    h = jnp.dot(x_ref[...], w1_ref[...], preferred_element_type=jnp.float32)
    hs = h * dinv
    hs_ref[0] = hs[:, :DH]
    hs_ref[1] = hs[:, DH:]
    dinv_ref[...] = dinv


def _tc1(x, w1, hist):
    return pl.pallas_call(
        _tc1_body,
        grid=(G,),
        in_specs=[
            pl.BlockSpec((R, D), lambda i: (i, 0)),
            pl.BlockSpec((D, D), lambda i: (0, 0)),
            pl.BlockSpec((NC, R, DO), lambda i: (0, i, 0)),
        ],
        out_specs=[
            pl.BlockSpec((NC, R, DH), lambda i: (0, i, 0)),
            pl.BlockSpec((R, 1), lambda i: (i, 0)),
        ],
        out_shape=[
            jax.ShapeDtypeStruct((NC, N, DH), jnp.float32),
            jax.ShapeDtypeStruct((N, 1), jnp.float32),
        ],
    )(x, w1, hist)


def _tc2_body(agg_ref, hs_ref, dinv_ref, b1_ref, w2_ref, h2s_ref):
    dinv = dinv_ref[...]
    a = jnp.concatenate([agg_ref[0] + hs_ref[0],
                         agg_ref[1] + hs_ref[1]], axis=1)
    out1 = a * dinv + b1_ref[...]
    r = jnp.maximum(out1, 0.0)
    h2 = jnp.dot(r, w2_ref[...], preferred_element_type=jnp.float32)
    h2s_ref[...] = h2 * dinv


def _tc2(agg, hs2, dinv, b1r, w2p):
    return pl.pallas_call(
        _tc2_body,
        grid=(G,),
        in_specs=[
            pl.BlockSpec((NC, R, DH), lambda i: (0, i, 0)),
            pl.BlockSpec((NC, R, DH), lambda i: (0, i, 0)),
            pl.BlockSpec((R, 1), lambda i: (i, 0)),
            pl.BlockSpec((1, D), lambda i: (0, 0)),
            pl.BlockSpec((D, DO), lambda i: (0, 0)),
        ],
        out_specs=pl.BlockSpec((R, DO), lambda i: (i, 0)),
        out_shape=jax.ShapeDtypeStruct((N, DO), jnp.float32),
    )(agg, hs2, dinv, b1r, w2p)


def _tc3_body(agg_ref, h2s_ref, dinv_ref, b2_ref, out_ref):
    a = agg_ref[0] + agg_ref[1] + h2s_ref[...]
    out_ref[...] = (a * dinv_ref[...])[:, :3] + b2_ref[...]


def _tc3(agg2, h2s, dinv, b2r):
    return pl.pallas_call(
        _tc3_body,
        grid=(G,),
        in_specs=[
            pl.BlockSpec((NC, R, DO), lambda i: (0, i, 0)),
            pl.BlockSpec((R, DO), lambda i: (i, 0)),
            pl.BlockSpec((R, 1), lambda i: (i, 0)),
            pl.BlockSpec((1, 3), lambda i: (0, 0)),
        ],
        out_specs=pl.BlockSpec((R, 3), lambda i: (i, 0)),
        out_shape=jax.ShapeDtypeStruct((N, 3), jnp.float32),
    )(agg2, h2s, dinv, b2r)


def kernel(x, edge_index, W1, b1, W2, b2):
    f32 = jnp.float32
    src = edge_index[0]
    dst = edge_index[1]
    src_p = jnp.concatenate([src, jnp.zeros((EP - E,), jnp.int32)])
    dst_p = jnp.concatenate([dst, jnp.full((EP - E,), TRASH, jnp.int32)])
    srcs = src_p.reshape(NW, CPT, CHUNK)
    dsts = dst_p.reshape(NW, CPT, CHUNK)
    srcs_b = src_p.reshape(NS, CPT2, CHUNK)
    dsts_b = dst_p.reshape(NS, CPT2, CHUNK)
    ones_r = jnp.ones((CHUNK, DO), f32)
    zeros_b = jnp.zeros((RPT, DH), f32)
    zeros_c = jnp.zeros((RPT, DO), f32)
    w2p = jnp.concatenate([W2, jnp.zeros((D, DO - W2.shape[1]), f32)], axis=1)
    b1r = b1.reshape(1, D)
    b2r = b2.reshape(1, 3)

    hist = _deg_kernel(dsts, ones_r, zeros_c)

    hs2, dinv = _tc1(x, W1, hist)

    agg = _agg128(hs2, srcs_b, dsts_b, zeros_b)

    h2s = _tc2(agg, hs2, dinv, b1r, w2p)

    agg2 = _agg16(h2s, srcs, dsts, zeros_c)

    return _tc3(agg2, h2s, dinv, b2r)


# speedup vs baseline: 1.0393x; 1.0393x over previous
import functools

import jax
import jax.numpy as jnp
from jax import lax
from jax.experimental import pallas as pl
from jax.experimental.pallas import tpu as pltpu
from jax.experimental.pallas import tpu_sc as plsc

N = 10000
E = 320000
D = 128
DH = D // 2
DO = 16
NN = 10016
NC = 2
NS = 16
NW = NC * NS
CHUNK = 128
CPT = 80
CPT2 = 160
EP = NW * CPT * CHUNK
TRASH = N
RPT = NN // NS
NBUF = 5
G = 5
R = N // G

_mesh = plsc.VectorSubcoreMesh(core_axis_name="c", subcore_axis_name="s")


@functools.partial(
    pl.kernel,
    out_type=jax.ShapeDtypeStruct((NC, NN, DO), jnp.float32),
    mesh=_mesh,
    scratch_types=[
        pltpu.VMEM((CPT, CHUNK), jnp.int32),
        pltpu.VMEM((CHUNK, DO), jnp.float32),
        pltpu.SemaphoreType.DMA,
        pltpu.SemaphoreType.DMA,
        pltpu.SemaphoreType.DMA,
        pltpu.SemaphoreType.DMA,
        pltpu.VMEM_SHARED((NN, DO), jnp.float32),
    ],
    compiler_params=pltpu.CompilerParams(use_tc_tiling_on_sc=False),
)
def _deg_kernel(dsts_hbm, ones_hbm, zeros_hbm, hist_out,
                dst_v, ones_v, s0, s1, s2, s3, hist_s):
    cid = lax.axis_index("c")
    sid = lax.axis_index("s")
    wid = cid * NS + sid
    ssem = (s0, s1, s2, s3)
    pltpu.sync_copy(dsts_hbm.at[wid], dst_v)
    pltpu.sync_copy(ones_hbm, ones_v)
    pltpu.sync_copy(zeros_hbm, hist_s.at[pl.ds(sid * RPT, RPT)])
    plsc.subcore_barrier()

    def body(i, carry):
        for b in range(4):
            j = 4 * i + b

            @pl.when(i > 0)
            def _():
                pltpu.make_async_copy(ones_v, hist_s.at[dst_v.at[j - 4]],
                                      ssem[b]).wait()

            pltpu.async_copy(ones_v, hist_s.at[dst_v.at[j]], ssem[b],
                             add=True)
        return carry

    lax.fori_loop(0, CPT // 4, body, 0)
    for b in range(4):
        pltpu.make_async_copy(ones_v, hist_s.at[dst_v.at[CPT - 4 + b]],
                              ssem[b]).wait()
    plsc.subcore_barrier()
    pltpu.sync_copy(hist_s.at[pl.ds(sid * RPT, RPT)],
                    hist_out.at[cid].at[pl.ds(sid * RPT, RPT)])


def _pipeline(table, src_v, dst_v, bufs, gsem, ssem, acc, n_chunks):
    def start_gather(j, b):
        pltpu.async_copy(table.at[src_v.at[j]], bufs[b], gsem[b])

    def wait_gather(j, b):
        pltpu.make_async_copy(table.at[src_v.at[j]], bufs[b], gsem[b]).wait()

    def start_scatter(j, b):
        pltpu.async_copy(bufs[b], acc.at[dst_v.at[j]], ssem[b], add=True)

    def wait_scatter(j, b):
        pltpu.make_async_copy(bufs[b], acc.at[dst_v.at[j]], ssem[b]).wait()

    ngroups = n_chunks // NBUF
    for b in range(NBUF):
        start_gather(b, b)

    def body(g, carry):
        c0 = NBUF * g
        for b in range(NBUF):
            wait_gather(c0 + b, b)
            start_scatter(c0 + b, b)
        for b in range(NBUF):
            wait_scatter(c0 + b, b)

            @pl.when(g < ngroups - 1)
            def _():
                start_gather(c0 + NBUF + b, b)

        return carry

    lax.fori_loop(0, ngroups, body, 0)


def _agg_scratch(cpt, d):
    return (
        [pltpu.VMEM((cpt, CHUNK), jnp.int32)] * 2
        + [pltpu.VMEM((CHUNK, d), jnp.float32)] * NBUF
        + [pltpu.SemaphoreType.DMA] * (2 * NBUF)
        + [pltpu.VMEM_SHARED((NN, d), jnp.float32)]
    )


_LASTB = N - (NS - 1) * RPT
CB = 64
NB2 = 10
CPTB = EP // (NS * CB)
_NGRP = CPTB // NB2
_NHALF = _NGRP // 2


@functools.partial(
    pl.kernel,
    out_type=jax.ShapeDtypeStruct((NC, NN, DH), jnp.float32),
    mesh=_mesh,
    scratch_types=(
        [pltpu.VMEM((NB2, CB), jnp.int32)] * 4
        + [pltpu.VMEM((CB, DH), jnp.float32)] * NB2
        + [pltpu.SemaphoreType.DMA] * (2 * NB2 + 4)
        + [pltpu.VMEM_SHARED((NN, DH), jnp.float32)]
        + [pltpu.VMEM_SHARED((NN, DH), jnp.float32)]
    ),
    compiler_params=pltpu.CompilerParams(use_tc_tiling_on_sc=False),
)
def _agg128(hs2_hbm, srcs_hbm, dsts_hbm, zeros_hbm, out_hbm,
            w0s, w0d, w1s, w1d, *rest):
    bufs = rest[:NB2]
    gsem = rest[NB2:2 * NB2]
    ssem = rest[2 * NB2:3 * NB2]
    is0, id0, is1, id1 = rest[3 * NB2:3 * NB2 + 4]
    table_s, acc = rest[-2], rest[-1]
    cid = lax.axis_index("c")
    sid = lax.axis_index("s")
    my_src = srcs_hbm.at[sid]
    my_dst = dsts_hbm.at[sid]

    def swin(g):
        return my_src.at[pl.ds(g * NB2, NB2)]

    def dwin(g):
        return my_dst.at[pl.ds(g * NB2, NB2)]

    def phase1(cs, cd, g):
        for b in range(NB2):
            pltpu.make_async_copy(table_s.at[cs.at[b]], bufs[b],
                                  gsem[b]).wait()
            pltpu.async_copy(bufs[b], acc.at[cd.at[b]], ssem[b], add=True)

    def phase2(cd, ns, g):
        for b in range(NB2):
            pltpu.make_async_copy(bufs[b], acc.at[cd.at[b]], ssem[b]).wait()

            @pl.when(g < _NGRP - 1)
            def _():
                pltpu.async_copy(table_s.at[ns.at[b]], bufs[b], gsem[b])

    pltpu.sync_copy(zeros_hbm, acc.at[pl.ds(sid * RPT, RPT)])

    @pl.when(sid < NS - 1)
    def _():
        pltpu.sync_copy(hs2_hbm.at[cid].at[pl.ds(sid * RPT, RPT)],
                        table_s.at[pl.ds(sid * RPT, RPT)])

    @pl.when(sid == NS - 1)
    def _():
        pltpu.sync_copy(hs2_hbm.at[cid].at[pl.ds((NS - 1) * RPT, _LASTB)],
                        table_s.at[pl.ds((NS - 1) * RPT, _LASTB)])

    pltpu.sync_copy(swin(0), w0s)
    pltpu.sync_copy(dwin(0), w0d)
    pltpu.async_copy(swin(1), w1s, is1)
    pltpu.async_copy(dwin(1), w1d, id1)
    plsc.subcore_barrier()
    for b in range(NB2):
        pltpu.async_copy(table_s.at[w0s.at[b]], bufs[b], gsem[b])

    def body(i, carry):
        g0 = 2 * i
        g1 = g0 + 1
        pltpu.make_async_copy(swin(g1), w1s, is1).wait()
        pltpu.make_async_copy(dwin(g1), w1d, id1).wait()
        phase1(w0s, w0d, g0)

        @pl.when(i < _NHALF - 1)
        def _():
            pltpu.async_copy(swin(g0 + 2), w0s, is0)

        phase2(w0d, w1s, g0)

        @pl.when(i < _NHALF - 1)
        def _():
            pltpu.async_copy(dwin(g0 + 2), w0d, id0)
            pltpu.make_async_copy(swin(g0 + 2), w0s, is0).wait()

        phase1(w1s, w1d, g1)

        @pl.when(i < _NHALF - 1)
        def _():
            pltpu.async_copy(swin(g1 + 2), w1s, is1)

        phase2(w1d, w0s, g1)

        @pl.when(i < _NHALF - 1)
        def _():
            pltpu.async_copy(dwin(g1 + 2), w1d, id1)
            pltpu.make_async_copy(dwin(g0 + 2), w0d, id0).wait()

        return carry

    lax.fori_loop(0, _NHALF, body, 0)
    plsc.subcore_barrier()
    pltpu.sync_copy(acc.at[pl.ds(sid * RPT, RPT)],
                    out_hbm.at[cid].at[pl.ds(sid * RPT, RPT)])


_LAST = N - (NS - 1) * RPT


@functools.partial(
    pl.kernel,
    out_type=jax.ShapeDtypeStruct((NC, NN, DO), jnp.float32),
    mesh=_mesh,
    scratch_types=_agg_scratch(CPT, DO)
    + [pltpu.VMEM_SHARED((NN, DO), jnp.float32)],
    compiler_params=pltpu.CompilerParams(use_tc_tiling_on_sc=False),
)
def _agg16(h2s_hbm, srcs_hbm, dsts_hbm, zeros_hbm, out_hbm,
           src_v, dst_v, *rest):
    bufs, gsem, ssem, acc, table_s = (rest[:NBUF], rest[NBUF:2 * NBUF],
                                      rest[2 * NBUF:3 * NBUF], rest[-2],
                                      rest[-1])
    cid = lax.axis_index("c")
    sid = lax.axis_index("s")
    wid = cid * NS + sid
    pltpu.sync_copy(srcs_hbm.at[wid], src_v)
    pltpu.sync_copy(dsts_hbm.at[wid], dst_v)
    pltpu.sync_copy(zeros_hbm, acc.at[pl.ds(sid * RPT, RPT)])

    @pl.when(sid < NS - 1)
    def _():
        pltpu.sync_copy(h2s_hbm.at[pl.ds(sid * RPT, RPT)],
                        table_s.at[pl.ds(sid * RPT, RPT)])

    @pl.when(sid == NS - 1)
    def _():
        pltpu.sync_copy(h2s_hbm.at[pl.ds((NS - 1) * RPT, _LAST)],
                        table_s.at[pl.ds((NS - 1) * RPT, _LAST)])

    plsc.subcore_barrier()
    _pipeline(table_s, src_v, dst_v, bufs, gsem, ssem, acc, CPT)
    plsc.subcore_barrier()
    pltpu.sync_copy(acc.at[pl.ds(sid * RPT, RPT)],
                    out_hbm.at[cid].at[pl.ds(sid * RPT, RPT)])


def _tc1_body(x_ref, w1_ref, hist_ref, hs_ref, dinv_ref):
    deg = hist_ref[0, :, 0:1] + hist_ref[1, :, 0:1]
    dinv = lax.rsqrt(deg + 1.0)
    h = jnp.dot(x_ref[...], w1_ref[...], preferred_element_type=jnp.float32)
    hs = h * dinv
    hs_ref[0] = hs[:, :DH]
    hs_ref[1] = hs[:, DH:]
    dinv_ref[...] = dinv


def _tc1(x, w1, hist):
    return pl.pallas_call(
        _tc1_body,
        grid=(G,),
        in_specs=[
            pl.BlockSpec((R, D), lambda i: (i, 0)),
            pl.BlockSpec((D, D), lambda i: (0, 0)),
            pl.BlockSpec((NC, R, DO), lambda i: (0, i, 0)),
        ],
        out_specs=[
            pl.BlockSpec((NC, R, DH), lambda i: (0, i, 0)),
            pl.BlockSpec((R, 1), lambda i: (i, 0)),
        ],
        out_shape=[
            jax.ShapeDtypeStruct((NC, N, DH), jnp.float32),
            jax.ShapeDtypeStruct((N, 1), jnp.float32),
        ],
    )(x, w1, hist)


def _tc2_body(agg_ref, hs_ref, dinv_ref, b1_ref, w2_ref, h2s_ref):
    dinv = dinv_ref[...]
    a = jnp.concatenate([agg_ref[0] + hs_ref[0],
                         agg_ref[1] + hs_ref[1]], axis=1)
    out1 = a * dinv + b1_ref[...]
    r = jnp.maximum(out1, 0.0)
    h2 = jnp.dot(r, w2_ref[...], preferred_element_type=jnp.float32)
    h2s_ref[...] = h2 * dinv


def _tc2(agg, hs2, dinv, b1r, w2p):
    return pl.pallas_call(
        _tc2_body,
        grid=(G,),
        in_specs=[
            pl.BlockSpec((NC, R, DH), lambda i: (0, i, 0)),
            pl.BlockSpec((NC, R, DH), lambda i: (0, i, 0)),
            pl.BlockSpec((R, 1), lambda i: (i, 0)),
            pl.BlockSpec((1, D), lambda i: (0, 0)),
            pl.BlockSpec((D, DO), lambda i: (0, 0)),
        ],
        out_specs=pl.BlockSpec((R, DO), lambda i: (i, 0)),
        out_shape=jax.ShapeDtypeStruct((N, DO), jnp.float32),
    )(agg, hs2, dinv, b1r, w2p)


def _tc3_body(agg_ref, h2s_ref, dinv_ref, b2_ref, out_ref):
    a = agg_ref[0] + agg_ref[1] + h2s_ref[...]
    out_ref[...] = (a * dinv_ref[...])[:, :3] + b2_ref[...]


def _tc3(agg2, h2s, dinv, b2r):
    return pl.pallas_call(
        _tc3_body,
        grid=(G,),
        in_specs=[
            pl.BlockSpec((NC, R, DO), lambda i: (0, i, 0)),
            pl.BlockSpec((R, DO), lambda i: (i, 0)),
            pl.BlockSpec((R, 1), lambda i: (i, 0)),
            pl.BlockSpec((1, 3), lambda i: (0, 0)),
        ],
        out_specs=pl.BlockSpec((R, 3), lambda i: (i, 0)),
        out_shape=jax.ShapeDtypeStruct((N, 3), jnp.float32),
    )(agg2, h2s, dinv, b2r)


def kernel(x, edge_index, W1, b1, W2, b2):
    f32 = jnp.float32
    src = edge_index[0]
    dst = edge_index[1]
    src_p = jnp.concatenate([src, jnp.zeros((EP - E,), jnp.int32)])
    dst_p = jnp.concatenate([dst, jnp.full((EP - E,), TRASH, jnp.int32)])
    srcs = src_p.reshape(NW, CPT, CHUNK)
    dsts = dst_p.reshape(NW, CPT, CHUNK)
    srcs_b = src_p.reshape(NS, CPTB, CB)
    dsts_b = dst_p.reshape(NS, CPTB, CB)
    ones_r = jnp.ones((CHUNK, DO), f32)
    zeros_b = jnp.zeros((RPT, DH), f32)
    zeros_c = jnp.zeros((RPT, DO), f32)
    w2p = jnp.concatenate([W2, jnp.zeros((D, DO - W2.shape[1]), f32)], axis=1)
    b1r = b1.reshape(1, D)
    b2r = b2.reshape(1, 3)

    hist = _deg_kernel(dsts, ones_r, zeros_c)

    hs2, dinv = _tc1(x, W1, hist)

    agg = _agg128(hs2, srcs_b, dsts_b, zeros_b)

    h2s = _tc2(agg, hs2, dinv, b1r, w2p)

    agg2 = _agg16(h2s, srcs, dsts, zeros_c)

    return _tc3(agg2, h2s, dinv, b2r)


# speedup vs baseline: 1.0414x; 1.0020x over previous
import functools

import jax
import jax.numpy as jnp
from jax import lax
from jax.experimental import pallas as pl
from jax.experimental.pallas import tpu as pltpu
from jax.experimental.pallas import tpu_sc as plsc

N = 10000
E = 320000
D = 128
DH = D // 2
DO = 16
NN = 10016
NC = 2
NS = 16
NW = NC * NS
CHUNK = 128
CPT = 80
CPT2 = 160
EP = NW * CPT * CHUNK
TRASH = N
RPT = NN // NS
NBUF = 5
G = 5
R = N // G

_mesh = plsc.VectorSubcoreMesh(core_axis_name="c", subcore_axis_name="s")


@functools.partial(
    pl.kernel,
    out_type=jax.ShapeDtypeStruct((NC, NN, DO), jnp.float32),
    mesh=_mesh,
    scratch_types=[
        pltpu.VMEM((CPT, CHUNK), jnp.int32),
        pltpu.VMEM((CHUNK, DO), jnp.float32),
        pltpu.SemaphoreType.DMA,
        pltpu.SemaphoreType.DMA,
        pltpu.SemaphoreType.DMA,
        pltpu.SemaphoreType.DMA,
        pltpu.VMEM_SHARED((NN, DO), jnp.float32),
    ],
    compiler_params=pltpu.CompilerParams(use_tc_tiling_on_sc=False),
)
def _deg_kernel(dsts_hbm, ones_hbm, zeros_hbm, hist_out,
                dst_v, ones_v, s0, s1, s2, s3, hist_s):
    cid = lax.axis_index("c")
    sid = lax.axis_index("s")
    wid = cid * NS + sid
    ssem = (s0, s1, s2, s3)
    pltpu.sync_copy(dsts_hbm.at[wid], dst_v)
    pltpu.sync_copy(ones_hbm, ones_v)
    pltpu.sync_copy(zeros_hbm, hist_s.at[pl.ds(sid * RPT, RPT)])
    plsc.subcore_barrier()

    def body(i, carry):
        for b in range(4):
            j = 4 * i + b

            @pl.when(i > 0)
            def _():
                pltpu.make_async_copy(ones_v, hist_s.at[dst_v.at[j - 4]],
                                      ssem[b]).wait()

            pltpu.async_copy(ones_v, hist_s.at[dst_v.at[j]], ssem[b],
                             add=True)
        return carry

    lax.fori_loop(0, CPT // 4, body, 0)
    for b in range(4):
        pltpu.make_async_copy(ones_v, hist_s.at[dst_v.at[CPT - 4 + b]],
                              ssem[b]).wait()
    plsc.subcore_barrier()
    pltpu.sync_copy(hist_s.at[pl.ds(sid * RPT, RPT)],
                    hist_out.at[cid].at[pl.ds(sid * RPT, RPT)])


def _pipeline(table, src_v, dst_v, bufs, gsem, ssem, acc, n_chunks):
    nbuf = len(bufs)

    def start_gather(j, b):
        pltpu.async_copy(table.at[src_v.at[j]], bufs[b], gsem[b])

    def wait_gather(j, b):
        pltpu.make_async_copy(table.at[src_v.at[j]], bufs[b], gsem[b]).wait()

    def start_scatter(j, b):
        pltpu.async_copy(bufs[b], acc.at[dst_v.at[j]], ssem[b], add=True)

    def wait_scatter(j, b):
        pltpu.make_async_copy(bufs[b], acc.at[dst_v.at[j]], ssem[b]).wait()

    ngroups = n_chunks // nbuf
    for b in range(nbuf):
        start_gather(b, b)

    def body(g, carry):
        c0 = nbuf * g
        for b in range(nbuf):
            wait_gather(c0 + b, b)
            start_scatter(c0 + b, b)
        for b in range(nbuf):
            wait_scatter(c0 + b, b)

            @pl.when(g < ngroups - 1)
            def _():
                start_gather(c0 + nbuf + b, b)

        return carry

    lax.fori_loop(0, ngroups, body, 0)


def _agg_scratch(cpt, chunk, nbuf, d):
    return (
        [pltpu.VMEM((cpt, chunk), jnp.int32)] * 2
        + [pltpu.VMEM((chunk, d), jnp.float32)] * nbuf
        + [pltpu.SemaphoreType.DMA] * (2 * nbuf)
        + [pltpu.VMEM_SHARED((NN, d), jnp.float32)]
    )


_LASTB = N - (NS - 1) * RPT
CB = 64
NB2 = 10
CPTB = EP // (NS * CB)
_NGRP = CPTB // NB2
_NHALF = _NGRP // 2


@functools.partial(
    pl.kernel,
    out_type=jax.ShapeDtypeStruct((NC, NN, DH), jnp.float32),
    mesh=_mesh,
    scratch_types=(
        [pltpu.VMEM((NB2, CB), jnp.int32)] * 4
        + [pltpu.VMEM((CB, DH), jnp.float32)] * NB2
        + [pltpu.SemaphoreType.DMA] * (2 * NB2 + 4)
        + [pltpu.VMEM_SHARED((NN, DH), jnp.float32)]
        + [pltpu.VMEM_SHARED((NN, DH), jnp.float32)]
    ),
    compiler_params=pltpu.CompilerParams(use_tc_tiling_on_sc=False),
)
def _agg128(hs2_hbm, srcs_hbm, dsts_hbm, zeros_hbm, out_hbm,
            w0s, w0d, w1s, w1d, *rest):
    bufs = rest[:NB2]
    gsem = rest[NB2:2 * NB2]
    ssem = rest[2 * NB2:3 * NB2]
    is0, id0, is1, id1 = rest[3 * NB2:3 * NB2 + 4]
    table_s, acc = rest[-2], rest[-1]
    cid = lax.axis_index("c")
    sid = lax.axis_index("s")
    my_src = srcs_hbm.at[sid]
    my_dst = dsts_hbm.at[sid]

    def swin(g):
        return my_src.at[pl.ds(g * NB2, NB2)]

    def dwin(g):
        return my_dst.at[pl.ds(g * NB2, NB2)]

    def phase1(cs, cd, g):
        for b in range(NB2):
            pltpu.make_async_copy(table_s.at[cs.at[b]], bufs[b],
                                  gsem[b]).wait()
            pltpu.async_copy(bufs[b], acc.at[cd.at[b]], ssem[b], add=True)

    def phase2(cd, ns, g):
        for b in range(NB2):
            pltpu.make_async_copy(bufs[b], acc.at[cd.at[b]], ssem[b]).wait()

            @pl.when(g < _NGRP - 1)
            def _():
                pltpu.async_copy(table_s.at[ns.at[b]], bufs[b], gsem[b])

    pltpu.sync_copy(zeros_hbm, acc.at[pl.ds(sid * RPT, RPT)])

    @pl.when(sid < NS - 1)
    def _():
        pltpu.sync_copy(hs2_hbm.at[cid].at[pl.ds(sid * RPT, RPT)],
                        table_s.at[pl.ds(sid * RPT, RPT)])

    @pl.when(sid == NS - 1)
    def _():
        pltpu.sync_copy(hs2_hbm.at[cid].at[pl.ds((NS - 1) * RPT, _LASTB)],
                        table_s.at[pl.ds((NS - 1) * RPT, _LASTB)])

    pltpu.sync_copy(swin(0), w0s)
    pltpu.sync_copy(dwin(0), w0d)
    pltpu.async_copy(swin(1), w1s, is1)
    pltpu.async_copy(dwin(1), w1d, id1)
    plsc.subcore_barrier()
    for b in range(NB2):
        pltpu.async_copy(table_s.at[w0s.at[b]], bufs[b], gsem[b])

    def body(i, carry):
        g0 = 2 * i
        g1 = g0 + 1
        pltpu.make_async_copy(swin(g1), w1s, is1).wait()
        pltpu.make_async_copy(dwin(g1), w1d, id1).wait()
        phase1(w0s, w0d, g0)

        @pl.when(i < _NHALF - 1)
        def _():
            pltpu.async_copy(swin(g0 + 2), w0s, is0)

        phase2(w0d, w1s, g0)

        @pl.when(i < _NHALF - 1)
        def _():
            pltpu.async_copy(dwin(g0 + 2), w0d, id0)
            pltpu.make_async_copy(swin(g0 + 2), w0s, is0).wait()

        phase1(w1s, w1d, g1)

        @pl.when(i < _NHALF - 1)
        def _():
            pltpu.async_copy(swin(g1 + 2), w1s, is1)

        phase2(w1d, w0s, g1)

        @pl.when(i < _NHALF - 1)
        def _():
            pltpu.async_copy(dwin(g1 + 2), w1d, id1)
            pltpu.make_async_copy(dwin(g0 + 2), w0d, id0).wait()

        return carry

    lax.fori_loop(0, _NHALF, body, 0)
    plsc.subcore_barrier()
    pltpu.sync_copy(acc.at[pl.ds(sid * RPT, RPT)],
                    out_hbm.at[cid].at[pl.ds(sid * RPT, RPT)])


_LAST = N - (NS - 1) * RPT


CPTC = EP // (NW * CB)


@functools.partial(
    pl.kernel,
    out_type=jax.ShapeDtypeStruct((NC, NN, DO), jnp.float32),
    mesh=_mesh,
    scratch_types=_agg_scratch(CPTC, CB, NB2, DO)
    + [pltpu.VMEM_SHARED((NN, DO), jnp.float32)],
    compiler_params=pltpu.CompilerParams(use_tc_tiling_on_sc=False),
)
def _agg16(h2s_hbm, srcs_hbm, dsts_hbm, zeros_hbm, out_hbm,
           src_v, dst_v, *rest):
    bufs, gsem, ssem, acc, table_s = (rest[:NB2], rest[NB2:2 * NB2],
                                      rest[2 * NB2:3 * NB2], rest[-2],
                                      rest[-1])
    cid = lax.axis_index("c")
    sid = lax.axis_index("s")
    wid = cid * NS + sid
    pltpu.sync_copy(srcs_hbm.at[wid], src_v)
    pltpu.sync_copy(dsts_hbm.at[wid], dst_v)
    pltpu.sync_copy(zeros_hbm, acc.at[pl.ds(sid * RPT, RPT)])

    @pl.when(sid < NS - 1)
    def _():
        pltpu.sync_copy(h2s_hbm.at[pl.ds(sid * RPT, RPT)],
                        table_s.at[pl.ds(sid * RPT, RPT)])

    @pl.when(sid == NS - 1)
    def _():
        pltpu.sync_copy(h2s_hbm.at[pl.ds((NS - 1) * RPT, _LAST)],
                        table_s.at[pl.ds((NS - 1) * RPT, _LAST)])

    plsc.subcore_barrier()
    _pipeline(table_s, src_v, dst_v, bufs, gsem, ssem, acc, CPTC)
    plsc.subcore_barrier()
    pltpu.sync_copy(acc.at[pl.ds(sid * RPT, RPT)],
                    out_hbm.at[cid].at[pl.ds(sid * RPT, RPT)])


def _tc1_body(x_ref, w1_ref, hist_ref, hs_ref, dinv_ref):
    deg = hist_ref[0, :, 0:1] + hist_ref[1, :, 0:1]
    dinv = lax.rsqrt(deg + 1.0)
    h = jnp.dot(x_ref[...], w1_ref[...], preferred_element_type=jnp.float32)
    hs = h * dinv
    hs_ref[0] = hs[:, :DH]
    hs_ref[1] = hs[:, DH:]
    dinv_ref[...] = dinv


def _tc1(x, w1, hist):
    return pl.pallas_call(
        _tc1_body,
        grid=(G,),
        in_specs=[
            pl.BlockSpec((R, D), lambda i: (i, 0)),
            pl.BlockSpec((D, D), lambda i: (0, 0)),
            pl.BlockSpec((NC, R, DO), lambda i: (0, i, 0)),
        ],
        out_specs=[
            pl.BlockSpec((NC, R, DH), lambda i: (0, i, 0)),
            pl.BlockSpec((R, 1), lambda i: (i, 0)),
        ],
        out_shape=[
            jax.ShapeDtypeStruct((NC, N, DH), jnp.float32),
            jax.ShapeDtypeStruct((N, 1), jnp.float32),
        ],
    )(x, w1, hist)


def _tc2_body(agg_ref, hs_ref, dinv_ref, b1_ref, w2_ref, h2s_ref):
    dinv = dinv_ref[...]
    a = jnp.concatenate([agg_ref[0] + hs_ref[0],
                         agg_ref[1] + hs_ref[1]], axis=1)
    out1 = a * dinv + b1_ref[...]
    r = jnp.maximum(out1, 0.0)
    h2 = jnp.dot(r, w2_ref[...], preferred_element_type=jnp.float32)
    h2s_ref[...] = h2 * dinv


def _tc2(agg, hs2, dinv, b1r, w2p):
    return pl.pallas_call(
        _tc2_body,
        grid=(G,),
        in_specs=[
            pl.BlockSpec((NC, R, DH), lambda i: (0, i, 0)),
            pl.BlockSpec((NC, R, DH), lambda i: (0, i, 0)),
            pl.BlockSpec((R, 1), lambda i: (i, 0)),
            pl.BlockSpec((1, D), lambda i: (0, 0)),
            pl.BlockSpec((D, DO), lambda i: (0, 0)),
        ],
        out_specs=pl.BlockSpec((R, DO), lambda i: (i, 0)),
        out_shape=jax.ShapeDtypeStruct((N, DO), jnp.float32),
    )(agg, hs2, dinv, b1r, w2p)


def _tc3_body(agg_ref, h2s_ref, dinv_ref, b2_ref, out_ref):
    a = agg_ref[0] + agg_ref[1] + h2s_ref[...]
    out_ref[...] = (a * dinv_ref[...])[:, :3] + b2_ref[...]


def _tc3(agg2, h2s, dinv, b2r):
    return pl.pallas_call(
        _tc3_body,
        grid=(G,),
        in_specs=[
            pl.BlockSpec((NC, R, DO), lambda i: (0, i, 0)),
            pl.BlockSpec((R, DO), lambda i: (i, 0)),
            pl.BlockSpec((R, 1), lambda i: (i, 0)),
            pl.BlockSpec((1, 3), lambda i: (0, 0)),
        ],
        out_specs=pl.BlockSpec((R, 3), lambda i: (i, 0)),
        out_shape=jax.ShapeDtypeStruct((N, 3), jnp.float32),
    )(agg2, h2s, dinv, b2r)


def kernel(x, edge_index, W1, b1, W2, b2):
    f32 = jnp.float32
    src = edge_index[0]
    dst = edge_index[1]
    src_p = jnp.concatenate([src, jnp.zeros((EP - E,), jnp.int32)])
    dst_p = jnp.concatenate([dst, jnp.full((EP - E,), TRASH, jnp.int32)])
    srcs = src_p.reshape(NW, CPT, CHUNK)
    dsts = dst_p.reshape(NW, CPT, CHUNK)
    srcs_c = src_p.reshape(NW, CPTC, CB)
    dsts_c = dst_p.reshape(NW, CPTC, CB)
    srcs_b = src_p.reshape(NS, CPTB, CB)
    dsts_b = dst_p.reshape(NS, CPTB, CB)
    ones_r = jnp.ones((CHUNK, DO), f32)
    zeros_b = jnp.zeros((RPT, DH), f32)
    zeros_c = jnp.zeros((RPT, DO), f32)
    w2p = jnp.concatenate([W2, jnp.zeros((D, DO - W2.shape[1]), f32)], axis=1)
    b1r = b1.reshape(1, D)
    b2r = b2.reshape(1, 3)

    hist = _deg_kernel(dsts, ones_r, zeros_c)

    hs2, dinv = _tc1(x, W1, hist)

    agg = _agg128(hs2, srcs_b, dsts_b, zeros_b)

    h2s = _tc2(agg, hs2, dinv, b1r, w2p)

    agg2 = _agg16(h2s, srcs_c, dsts_c, zeros_c)

    return _tc3(agg2, h2s, dinv, b2r)
